# Initial kernel scaffold; baseline (speedup 1.0000x reference)
import jax
import jax.numpy as jnp
from jax.experimental import pallas as pl


def kernel(x, edge_index, edge_attr, pos_ind, neg_ind, W_e, b_e, W1, b1, W2, b2, thr):
    raise NotImplementedError("write your pallas kernel here")


# speedup vs baseline: 3.7924x; 3.7924x over previous
import jax
import jax.numpy as jnp
from jax import lax
from jax.experimental import pallas as pl
from jax.experimental.pallas import tpu as pltpu
from jax.experimental.pallas import tpu_sc as plsc

N = 10000
E = 320000
D = 128
DE = 64

NC = 2
NS = 16
ET = E // (NC * NS)
NVEC = ET // 16
FLAGN = 10016
NEED = 384
AGGR_ROWS = 10240
DUMP = N
LCAP = 5120
PHALF = LCAP * NS
CAP = 2 * PHALF
TCB = 512


def _compact_kernel(dst_h, src_h, needid_h, x_h, eattr2_h,
                    dstids_h, xsrc_h, attrc_h, xneed_h, ktot_h,
                    flag_v, dst_v, src_v,
                    loc_srcE, loc_dstE, loc_e2E,
                    loc_srcO, loc_dstO, loc_e2O,
                    needid_v, cntv, xrows_v, arows_v, cnt_smem, sem):
    c = lax.axis_index("c")
    s = lax.axis_index("s")
    base = (c * NS + s) * ET

    @pl.when(s == 0)
    def _():
        cnt_smem[0] = 0
        cnt_smem[1] = 0
    plsc.subcore_barrier()

    zero16 = jnp.zeros((16,), jnp.int32)
    dump16 = jnp.full((16,), DUMP, jnp.int32)

    def init_flag(i, _):
        flag_v[pl.ds(i * 16, 16)] = zero16
        return 0
    lax.fori_loop(0, FLAGN // 16, init_flag, 0)

    def init_loc(i, _):
        loc_srcE[pl.ds(i * 16, 16)] = zero16
        loc_e2E[pl.ds(i * 16, 16)] = zero16
        loc_dstE[pl.ds(i * 16, 16)] = dump16
        loc_srcO[pl.ds(i * 16, 16)] = zero16
        loc_e2O[pl.ds(i * 16, 16)] = zero16
        loc_dstO[pl.ds(i * 16, 16)] = dump16
        return 0
    lax.fori_loop(0, LCAP // 16, init_loc, 0)

    pltpu.sync_copy(needid_h, needid_v)
    pltpu.sync_copy(dst_h.at[pl.ds(pl.multiple_of(base, 16), ET)], dst_v)
    pltpu.sync_copy(src_h.at[pl.ds(pl.multiple_of(base, 16), ET)], src_v)

    one16 = jnp.ones((16,), jnp.int32)

    def flag_body(i, _):
        ids = needid_v[pl.ds(i * 16, 16)]
        plsc.store_scatter(flag_v, [ids], one16)
        return 0
    lax.fori_loop(0, NEED // 16, flag_body, 0)

    iota16 = lax.iota(jnp.int32, 16)
    lane_even = (iota16 & 1) == 0

    def scan_body(i, carry):
        wpe, wpo = carry
        d = dst_v[pl.ds(i * 16, 16)]
        f = plsc.load_gather(flag_v, [d])
        m = f > 0
        me = m & lane_even
        mo = m & (~lane_even)
        mie = me.astype(jnp.int32)
        mio = mo.astype(jnp.int32)
        pose = wpe + plsc.cumsum(mie) - 1
        poso = wpo + plsc.cumsum(mio) - 1
        sv = src_v[pl.ds(i * 16, 16)]
        e2 = (base + i * 16 + iota16) >> 1
        plsc.store_scatter(loc_srcE, [pose], sv, mask=me)
        plsc.store_scatter(loc_dstE, [pose], d, mask=me)
        plsc.store_scatter(loc_e2E, [pose], e2, mask=me)
        plsc.store_scatter(loc_srcO, [poso], sv, mask=mo)
        plsc.store_scatter(loc_dstO, [poso], d, mask=mo)
        plsc.store_scatter(loc_e2O, [poso], e2, mask=mo)
        return wpe + jnp.sum(mie), wpo + jnp.sum(mio)

    kte, kto = lax.fori_loop(0, NVEC, scan_body,
                             (jnp.int32(0), jnp.int32(0)))
    rows_e = ((kte + 127) // 128) * 128
    rows_o = ((kto + 127) // 128) * 128
    offE = plsc.fetch_and_add(cnt_smem.at[0], rows_e, subcore_id=0)
    offO = plsc.fetch_and_add(cnt_smem.at[1], rows_o, subcore_id=0)
    plsc.subcore_barrier()

    @pl.when(s == 0)
    def _():
        tot_e = cnt_smem[0]
        tot_o = cnt_smem[1]
        cntv[...] = jnp.where(iota16 == 0, tot_e,
                              jnp.where(iota16 == 1, tot_o, 0))
        pltpu.sync_copy(cntv, ktot_h.at[pl.ds(pl.multiple_of(c * 16, 16), 16)])

    def emit(loc_src, loc_dst, loc_e2, nrows, goff):
        def chunk_body(j, _):
            r = pl.multiple_of(goff + j * 128, 128)
            pltpu.sync_copy(loc_dst.at[pl.ds(j * 128, 128)],
                            dstids_h.at[pl.ds(pl.multiple_of(c * CAP + r, 128),
                                              128)])
            pltpu.async_copy(x_h.at[loc_src.at[pl.ds(j * 128, 128)]],
                             xrows_v, sem).wait()
            pltpu.sync_copy(xrows_v, xsrc_h.at[c, pl.ds(r, 128)])
            pltpu.async_copy(eattr2_h.at[loc_e2.at[pl.ds(j * 128, 128)]],
                             arows_v, sem).wait()
            pltpu.sync_copy(arows_v, attrc_h.at[c, pl.ds(r, 128)])
            return 0
        lax.fori_loop(0, nrows // 128, chunk_body, 0)

    emit(loc_srcE, loc_dstE, loc_e2E, rows_e, offE)
    emit(loc_srcO, loc_dstO, loc_e2O, rows_o, offO + PHALF)

    @pl.when((c == 0) & (s == 0))
    def _():
        def need_body(j, _):
            pltpu.async_copy(x_h.at[needid_v.at[pl.ds(j * 128, 128)]],
                             xrows_v, sem).wait()
            pltpu.sync_copy(xrows_v, xneed_h.at[pl.ds(j * 128, 128)])
            return 0
        lax.fori_loop(0, NEED // 128, need_body, 0)


def _edge_mlp_kernel(ktot_s, wu_v, be_v, xsrc_h, attrc_h, msg_h,
                     xbuf, abuf, mbuf, sem1, sem2, sem3):
    for c in range(NC):
        for par in range(2):
            nblk = (ktot_s[c * 16 + par] + (TCB - 1)) // TCB
            goff = par * PHALF

            def blk(j, _):
                r = goff + j * TCB
                cp1 = pltpu.make_async_copy(xsrc_h.at[c, pl.ds(r, TCB)],
                                            xbuf, sem1)
                cp2 = pltpu.make_async_copy(attrc_h.at[c, pl.ds(r, TCB)],
                                            abuf, sem2)
                cp1.start()
                cp2.start()
                cp1.wait()
                cp2.wait()
                ea = jnp.dot(abuf[...], wu_v[par],
                             preferred_element_type=jnp.float32) + be_v[...]
                mbuf[...] = jnp.maximum(xbuf[...] + ea, 0.0)
                cp3 = pltpu.make_async_copy(mbuf, msg_h.at[c, pl.ds(r, TCB)],
                                            sem3)
                cp3.start()
                cp3.wait()
                return 0
            lax.fori_loop(0, nblk, blk, 0)


def _scatter_kernel(msg_h, dstids_h, ktot_h, needid_h, aggrneed_h,
                    aggr_sh, idxb, mbuf, kv, needid_v, sem):
    c = lax.axis_index("c")
    s = lax.axis_index("s")

    zrow = jnp.zeros((16,), jnp.float32)

    def zb(i, _):
        mbuf[i // 8, pl.ds((i % 8) * 16, 16)] = zrow
        return 0
    lax.fori_loop(0, 128 * 8, zb, 0)

    rows_per_tile = AGGR_ROWS // NS

    def zs(j, _):
        pltpu.sync_copy(mbuf,
                        aggr_sh.at[pl.ds(s * rows_per_tile + j * 128, 128)])
        return 0
    lax.fori_loop(0, rows_per_tile // 128, zs, 0)
    plsc.subcore_barrier()

    pltpu.sync_copy(ktot_h.at[pl.ds(pl.multiple_of(c * 16, 16), 16)], kv)
    kvv = kv[pl.ds(0, 16)]
    kte = kvv[0]
    kto = kvv[1]
    nblk_e = (kte + 127) // 128
    nblk = nblk_e + (kto + 127) // 128
    myblk = (nblk - s + (NS - 1)) // NS

    def sb(t, _):
        b = s + NS * t
        r = pl.multiple_of(
            jnp.where(b < nblk_e, b * 128, PHALF + (b - nblk_e) * 128), 128)
        pltpu.sync_copy(
            dstids_h.at[pl.ds(pl.multiple_of(c * CAP + r, 128), 128)], idxb)
        pltpu.sync_copy(msg_h.at[c, pl.ds(r, 128)], mbuf)
        pltpu.sync_copy(mbuf, aggr_sh.at[idxb], add=True)
        return 0
    lax.fori_loop(0, myblk, sb, 0)
    plsc.subcore_barrier()

    @pl.when(s < NEED // 128)
    def _():
        pltpu.sync_copy(needid_h, needid_v)
        pltpu.async_copy(aggr_sh.at[needid_v.at[pl.ds(s * 128, 128)]],
                         mbuf, sem).wait()
        pltpu.sync_copy(mbuf, aggrneed_h.at[c, pl.ds(s * 128, 128)])


def _readout_kernel(xneed_ref, aggr_ref, w1t_ref, b1_ref, w2t_ref, b2_ref,
                    thr_ref, out_ref):
    h = xneed_ref[...] + aggr_ref[0] + aggr_ref[1]
    pre = jnp.maximum(jnp.dot(h, w1t_ref[...],
                              preferred_element_type=jnp.float32)
                      + b1_ref[...], 0.0)
    emb = jnp.dot(pre, w2t_ref[...],
                  preferred_element_type=jnp.float32) + b2_ref[...]
    nrm = jnp.sqrt(jnp.sum(emb * emb, axis=1, keepdims=True))
    embn = emb / jnp.maximum(nrm, 1e-12)
    center = embn[0:1, :]
    sc = jnp.sum(embn * center, axis=1, keepdims=True)
    r = lax.broadcasted_iota(jnp.int32, (NEED, 1), 0)
    posm = (r >= 8) & (r < 136)
    negm = (r >= 136) & (r < 264)
    pos_mean = jnp.sum(jnp.where(posm, sc, 0.0)) / 128.0
    neg_mean = jnp.sum(jnp.where(negm, sc, 0.0)) / 128.0
    thr = thr_ref[...]
    tn = thr / jnp.maximum(jnp.sqrt(jnp.sum(thr * thr)), 1e-12)
    thr_score = jnp.sum(center * tn)
    pos_loss = jnp.log(jnp.maximum(jax.nn.sigmoid(pos_mean - thr_score),
                                   1e-12))
    neg_loss = jnp.log(jnp.maximum(jax.nn.sigmoid(thr_score - neg_mean),
                                   1e-12))
    out_ref[...] = jnp.full((8, 128), -(pos_loss + neg_loss), jnp.float32)


_sc_mesh = plsc.VectorSubcoreMesh(core_axis_name="c", subcore_axis_name="s",
                                  num_cores=NC, num_subcores=NS)

_compact = pl.kernel(
    _compact_kernel,
    out_type=(
        jax.ShapeDtypeStruct((NC * CAP,), jnp.int32),
        jax.ShapeDtypeStruct((NC, CAP, D), jnp.float32),
        jax.ShapeDtypeStruct((NC, CAP, D), jnp.float32),
        jax.ShapeDtypeStruct((NEED, D), jnp.float32),
        jax.ShapeDtypeStruct((NC * 16,), jnp.int32),
    ),
    mesh=_sc_mesh,
    compiler_params=pltpu.CompilerParams(needs_layout_passes=False),
    scratch_types=(
        pltpu.VMEM((FLAGN,), jnp.int32),
        pltpu.VMEM((ET,), jnp.int32),
        pltpu.VMEM((ET,), jnp.int32),
        pltpu.VMEM((LCAP,), jnp.int32),
        pltpu.VMEM((LCAP,), jnp.int32),
        pltpu.VMEM((LCAP,), jnp.int32),
        pltpu.VMEM((LCAP,), jnp.int32),
        pltpu.VMEM((LCAP,), jnp.int32),
        pltpu.VMEM((LCAP,), jnp.int32),
        pltpu.VMEM((NEED,), jnp.int32),
        pltpu.VMEM((16,), jnp.int32),
        pltpu.VMEM((128, D), jnp.float32),
        pltpu.VMEM((128, D), jnp.float32),
        pltpu.SMEM((2,), jnp.int32),
        pltpu.SemaphoreType.DMA,
    ),
)

_scatter = pl.kernel(
    _scatter_kernel,
    out_type=jax.ShapeDtypeStruct((NC, NEED, D), jnp.float32),
    mesh=_sc_mesh,
    compiler_params=pltpu.CompilerParams(needs_layout_passes=False),
    scratch_types=(
        pltpu.VMEM_SHARED((AGGR_ROWS, D), jnp.float32),
        pltpu.VMEM((128,), jnp.int32),
        pltpu.VMEM((128, D), jnp.float32),
        pltpu.VMEM((16,), jnp.int32),
        pltpu.VMEM((NEED,), jnp.int32),
        pltpu.SemaphoreType.DMA,
    ),
)

_edge_mlp = pl.pallas_call(
    _edge_mlp_kernel,
    out_shape=jax.ShapeDtypeStruct((NC, CAP, D), jnp.float32),
    in_specs=[
        pl.BlockSpec(memory_space=pltpu.SMEM),
        pl.BlockSpec(memory_space=pltpu.VMEM),
        pl.BlockSpec(memory_space=pltpu.VMEM),
        pl.BlockSpec(memory_space=pltpu.HBM),
        pl.BlockSpec(memory_space=pltpu.HBM),
    ],
    out_specs=pl.BlockSpec(memory_space=pltpu.HBM),
    scratch_shapes=[
        pltpu.VMEM((TCB, D), jnp.float32),
        pltpu.VMEM((TCB, D), jnp.float32),
        pltpu.VMEM((TCB, D), jnp.float32),
        pltpu.SemaphoreType.DMA,
        pltpu.SemaphoreType.DMA,
        pltpu.SemaphoreType.DMA,
    ],
)

_readout = pl.pallas_call(
    _readout_kernel,
    out_shape=jax.ShapeDtypeStruct((8, 128), jnp.float32),
)


def kernel(x, edge_index, edge_attr, pos_ind, neg_ind, W_e, b_e, W1, b1, W2,
           b2, thr):
    src = edge_index[0].astype(jnp.int32)
    dst = edge_index[1].astype(jnp.int32)
    needid = jnp.concatenate([
        jnp.zeros((8,), jnp.int32),
        pos_ind.astype(jnp.int32),
        neg_ind.astype(jnp.int32),
        jnp.zeros((NEED - 264,), jnp.int32),
    ])
    eattr2 = edge_attr.reshape(E // 2, 2 * DE)
    dstids, xsrc, attrc, xneed, ktot = _compact(dst, src, needid, x, eattr2)
    wet = W_e.T
    zpad = jnp.zeros((DE, D), jnp.float32)
    wu = jnp.stack([jnp.concatenate([wet, zpad], axis=0),
                    jnp.concatenate([zpad, wet], axis=0)])
    msg = _edge_mlp(ktot, wu, b_e.reshape(1, D), xsrc, attrc)
    aggrneed = _scatter(msg, dstids, ktot, needid)
    out = _readout(xneed, aggrneed, W1.T, b1.reshape(1, D), W2.T,
                   b2.reshape(1, D), thr)
    return out[0, 0]


# speedup vs baseline: 5.8728x; 1.5486x over previous
import jax
import jax.numpy as jnp
from jax import lax
from jax.experimental import pallas as pl
from jax.experimental.pallas import tpu as pltpu
from jax.experimental.pallas import tpu_sc as plsc

N = 10000
E = 320000
D = 128
DE = 64

NC = 2
NS = 16
ET = E // (NC * NS)
NVEC = ET // 16
FLAGN = 10016
NEED = 384
AGGR_ROWS = 10240
DUMP = N
LCAP = 5120
PHALF = LCAP * NS
CAP = 2 * PHALF
TCB = 2048


def _compact_kernel(dst_h, src_h, needid_h, x_h, eattr2_h, zflag_h,
                    dstids_h, xsrc_h, attrc_h, xneed_h, ktot_h,
                    flag_v, dst_v, src_v,
                    loc_eidE, loc_eidO, loc_src, loc_dst, loc_e2,
                    needid_v, cntv, xrows_v, arows_v, cnt_smem, sem):
    c = lax.axis_index("c")
    s = lax.axis_index("s")
    base = (c * NS + s) * ET

    @pl.when(s == 0)
    def _():
        cnt_smem[0] = 0
        cnt_smem[1] = 0
    plsc.subcore_barrier()

    pltpu.sync_copy(zflag_h, flag_v)
    pltpu.sync_copy(needid_h, needid_v)
    pltpu.sync_copy(dst_h.at[pl.ds(pl.multiple_of(base, 16), ET)], dst_v)
    pltpu.sync_copy(src_h.at[pl.ds(pl.multiple_of(base, 16), ET)], src_v)

    one16 = jnp.ones((16,), jnp.int32)
    dump16 = jnp.full((16,), DUMP, jnp.int32)
    base16 = jnp.full((16,), base, jnp.int32)
    iota16 = lax.iota(jnp.int32, 16)
    lane_even = (iota16 & 1) == 0

    def flag_body(i, _):
        ids = needid_v[pl.ds(i * 16, 16)]
        plsc.store_scatter(flag_v, [ids], one16)
        return 0
    lax.fori_loop(0, NEED // 16, flag_body, 0)

    def scan_body(i, carry):
        wpe, wpo = carry
        d = dst_v[pl.ds(i * 16, 16)]
        f = plsc.load_gather(flag_v, [d])
        m = f > 0
        me = m & lane_even
        mo = m & (~lane_even)
        mie = me.astype(jnp.int32)
        mio = mo.astype(jnp.int32)
        pose = wpe + plsc.cumsum(mie) - 1
        poso = wpo + plsc.cumsum(mio) - 1
        eid = base16 + (i * 16) + iota16
        plsc.store_scatter(loc_eidE, [pose], eid, mask=me)
        plsc.store_scatter(loc_eidO, [poso], eid, mask=mo)
        return wpe + jnp.sum(mie), wpo + jnp.sum(mio)

    kte, kto = lax.fori_loop(0, NVEC, scan_body,
                             (jnp.int32(0), jnp.int32(0)))
    rows_e = ((kte + 127) // 128) * 128
    rows_o = ((kto + 127) // 128) * 128
    offE = plsc.fetch_and_add(cnt_smem.at[0], rows_e, subcore_id=0)
    offO = plsc.fetch_and_add(cnt_smem.at[1], rows_o, subcore_id=0)
    plsc.subcore_barrier()

    @pl.when(s == 0)
    def _():
        tot_e = cnt_smem[0]
        tot_o = cnt_smem[1]
        cntv[...] = jnp.where(iota16 == 0, tot_e,
                              jnp.where(iota16 == 1, tot_o, 0))
        pltpu.sync_copy(cntv, ktot_h.at[pl.ds(pl.multiple_of(c * 16, 16), 16)])

    def emit(loc_eid, k, nrows, goff):
        def pad_eid(j, _):
            pos = k + j * 16 + iota16
            plsc.store_scatter(loc_eid, [pos], base16, mask=pos < nrows)
            return 0
        lax.fori_loop(0, 8, pad_eid, 0)

        def gpass(j, _):
            ev = loc_eid[pl.ds(j * 16, 16)]
            lv = ev - base16
            loc_src[pl.ds(j * 16, 16)] = plsc.load_gather(src_v, [lv])
            loc_dst[pl.ds(j * 16, 16)] = plsc.load_gather(dst_v, [lv])
            loc_e2[pl.ds(j * 16, 16)] = ev >> 1
            return 0
        lax.fori_loop(0, nrows // 16, gpass, 0)

        def pad_dst(j, _):
            pos = k + j * 16 + iota16
            plsc.store_scatter(loc_dst, [pos], dump16, mask=pos < nrows)
            return 0
        lax.fori_loop(0, 8, pad_dst, 0)

        def chunk_body(j, _):
            r = pl.multiple_of(goff + j * 128, 128)
            pltpu.sync_copy(loc_dst.at[pl.ds(j * 128, 128)],
                            dstids_h.at[pl.ds(pl.multiple_of(c * CAP + r, 128),
                                              128)])
            pltpu.async_copy(x_h.at[loc_src.at[pl.ds(j * 128, 128)]],
                             xrows_v, sem).wait()
            pltpu.sync_copy(xrows_v, xsrc_h.at[c, pl.ds(r, 128)])
            pltpu.async_copy(eattr2_h.at[loc_e2.at[pl.ds(j * 128, 128)]],
                             arows_v, sem).wait()
            pltpu.sync_copy(arows_v, attrc_h.at[c, pl.ds(r, 128)])
            return 0
        lax.fori_loop(0, nrows // 128, chunk_body, 0)

    emit(loc_eidE, kte, rows_e, offE)
    emit(loc_eidO, kto, rows_o, offO + PHALF)

    @pl.when((c == 0) & (s == 0))
    def _():
        def need_body(j, _):
            pltpu.async_copy(x_h.at[needid_v.at[pl.ds(j * 128, 128)]],
                             xrows_v, sem).wait()
            pltpu.sync_copy(xrows_v, xneed_h.at[pl.ds(j * 128, 128)])
            return 0
        lax.fori_loop(0, NEED // 128, need_body, 0)


def _edge_mlp_kernel(ktot_s, wu_v, be_v, xsrc_h, attrc_h, msg_h,
                     xbuf, abuf, mbuf, sem1, sem2, sem3):
    for c in range(NC):
        for par in range(2):
            nblk = (ktot_s[c * 16 + par] + (TCB - 1)) // TCB
            goff = par * PHALF

            def blk(j, _):
                r = goff + j * TCB
                cp1 = pltpu.make_async_copy(xsrc_h.at[c, pl.ds(r, TCB)],
                                            xbuf, sem1)
                cp2 = pltpu.make_async_copy(attrc_h.at[c, pl.ds(r, TCB)],
                                            abuf, sem2)
                cp1.start()
                cp2.start()
                cp1.wait()
                cp2.wait()
                ea = jnp.dot(abuf[...], wu_v[par],
                             preferred_element_type=jnp.float32) + be_v[...]
                mbuf[...] = jnp.maximum(xbuf[...] + ea, 0.0)
                cp3 = pltpu.make_async_copy(mbuf, msg_h.at[c, pl.ds(r, TCB)],
                                            sem3)
                cp3.start()
                cp3.wait()
                return 0
            lax.fori_loop(0, nblk, blk, 0)


def _scatter_kernel(msg_h, dstids_h, ktot_h, needid_h, aggrneed_h,
                    aggr_sh, idxb, mbuf, kv, needid_v, sem):
    c = lax.axis_index("c")
    s = lax.axis_index("s")

    zrow = jnp.zeros((16,), jnp.float32)

    def zb(i, _):
        mbuf[i // 8, pl.ds((i % 8) * 16, 16)] = zrow
        return 0
    lax.fori_loop(0, 128 * 8, zb, 0)

    rows_per_tile = AGGR_ROWS // NS

    def zs(j, _):
        pltpu.sync_copy(mbuf,
                        aggr_sh.at[pl.ds(s * rows_per_tile + j * 128, 128)])
        return 0
    lax.fori_loop(0, rows_per_tile // 128, zs, 0)
    plsc.subcore_barrier()

    pltpu.sync_copy(ktot_h.at[pl.ds(pl.multiple_of(c * 16, 16), 16)], kv)
    kvv = kv[pl.ds(0, 16)]
    kte = kvv[0]
    kto = kvv[1]
    nblk_e = (kte + 127) // 128
    nblk = nblk_e + (kto + 127) // 128
    myblk = (nblk - s + (NS - 1)) // NS

    def sb(t, _):
        b = s + NS * t
        r = pl.multiple_of(
            jnp.where(b < nblk_e, b * 128, PHALF + (b - nblk_e) * 128), 128)
        pltpu.sync_copy(
            dstids_h.at[pl.ds(pl.multiple_of(c * CAP + r, 128), 128)], idxb)
        pltpu.sync_copy(msg_h.at[c, pl.ds(r, 128)], mbuf)
        pltpu.sync_copy(mbuf, aggr_sh.at[idxb], add=True)
        return 0
    lax.fori_loop(0, myblk, sb, 0)
    plsc.subcore_barrier()

    @pl.when(s < NEED // 128)
    def _():
        pltpu.sync_copy(needid_h, needid_v)
        pltpu.async_copy(aggr_sh.at[needid_v.at[pl.ds(s * 128, 128)]],
                         mbuf, sem).wait()
        pltpu.sync_copy(mbuf, aggrneed_h.at[c, pl.ds(s * 128, 128)])


def _readout_kernel(xneed_ref, aggr_ref, w1t_ref, b1_ref, w2t_ref, b2_ref,
                    thr_ref, out_ref):
    h = xneed_ref[...] + aggr_ref[0] + aggr_ref[1]
    pre = jnp.maximum(jnp.dot(h, w1t_ref[...],
                              preferred_element_type=jnp.float32)
                      + b1_ref[...], 0.0)
    emb = jnp.dot(pre, w2t_ref[...],
                  preferred_element_type=jnp.float32) + b2_ref[...]
    nrm = jnp.sqrt(jnp.sum(emb * emb, axis=1, keepdims=True))
    embn = emb / jnp.maximum(nrm, 1e-12)
    center = embn[0:1, :]
    sc = jnp.sum(embn * center, axis=1, keepdims=True)
    r = lax.broadcasted_iota(jnp.int32, (NEED, 1), 0)
    posm = (r >= 8) & (r < 136)
    negm = (r >= 136) & (r < 264)
    pos_mean = jnp.sum(jnp.where(posm, sc, 0.0)) / 128.0
    neg_mean = jnp.sum(jnp.where(negm, sc, 0.0)) / 128.0
    thr = thr_ref[...]
    tn = thr / jnp.maximum(jnp.sqrt(jnp.sum(thr * thr)), 1e-12)
    thr_score = jnp.sum(center * tn)
    pos_loss = jnp.log(jnp.maximum(jax.nn.sigmoid(pos_mean - thr_score),
                                   1e-12))
    neg_loss = jnp.log(jnp.maximum(jax.nn.sigmoid(thr_score - neg_mean),
                                   1e-12))
    out_ref[...] = jnp.full((8, 128), -(pos_loss + neg_loss), jnp.float32)


_sc_mesh = plsc.VectorSubcoreMesh(core_axis_name="c", subcore_axis_name="s",
                                  num_cores=NC, num_subcores=NS)

_compact = pl.kernel(
    _compact_kernel,
    out_type=(
        jax.ShapeDtypeStruct((NC * CAP,), jnp.int32),
        jax.ShapeDtypeStruct((NC, CAP, D), jnp.float32),
        jax.ShapeDtypeStruct((NC, CAP, D), jnp.float32),
        jax.ShapeDtypeStruct((NEED, D), jnp.float32),
        jax.ShapeDtypeStruct((NC * 16,), jnp.int32),
    ),
    mesh=_sc_mesh,
    compiler_params=pltpu.CompilerParams(needs_layout_passes=False),
    scratch_types=(
        pltpu.VMEM((FLAGN,), jnp.int32),
        pltpu.VMEM((ET,), jnp.int32),
        pltpu.VMEM((ET,), jnp.int32),
        pltpu.VMEM((LCAP,), jnp.int32),
        pltpu.VMEM((LCAP,), jnp.int32),
        pltpu.VMEM((LCAP,), jnp.int32),
        pltpu.VMEM((LCAP,), jnp.int32),
        pltpu.VMEM((LCAP,), jnp.int32),
        pltpu.VMEM((NEED,), jnp.int32),
        pltpu.VMEM((16,), jnp.int32),
        pltpu.VMEM((128, D), jnp.float32),
        pltpu.VMEM((128, D), jnp.float32),
        pltpu.SMEM((2,), jnp.int32),
        pltpu.SemaphoreType.DMA,
    ),
)

_scatter = pl.kernel(
    _scatter_kernel,
    out_type=jax.ShapeDtypeStruct((NC, NEED, D), jnp.float32),
    mesh=_sc_mesh,
    compiler_params=pltpu.CompilerParams(needs_layout_passes=False),
    scratch_types=(
        pltpu.VMEM_SHARED((AGGR_ROWS, D), jnp.float32),
        pltpu.VMEM((128,), jnp.int32),
        pltpu.VMEM((128, D), jnp.float32),
        pltpu.VMEM((16,), jnp.int32),
        pltpu.VMEM((NEED,), jnp.int32),
        pltpu.SemaphoreType.DMA,
    ),
)

_edge_mlp = pl.pallas_call(
    _edge_mlp_kernel,
    out_shape=jax.ShapeDtypeStruct((NC, CAP, D), jnp.float32),
    in_specs=[
        pl.BlockSpec(memory_space=pltpu.SMEM),
        pl.BlockSpec(memory_space=pltpu.VMEM),
        pl.BlockSpec(memory_space=pltpu.VMEM),
        pl.BlockSpec(memory_space=pltpu.HBM),
        pl.BlockSpec(memory_space=pltpu.HBM),
    ],
    out_specs=pl.BlockSpec(memory_space=pltpu.HBM),
    scratch_shapes=[
        pltpu.VMEM((TCB, D), jnp.float32),
        pltpu.VMEM((TCB, D), jnp.float32),
        pltpu.VMEM((TCB, D), jnp.float32),
        pltpu.SemaphoreType.DMA,
        pltpu.SemaphoreType.DMA,
        pltpu.SemaphoreType.DMA,
    ],
)

_readout = pl.pallas_call(
    _readout_kernel,
    out_shape=jax.ShapeDtypeStruct((8, 128), jnp.float32),
)


def kernel(x, edge_index, edge_attr, pos_ind, neg_ind, W_e, b_e, W1, b1, W2,
           b2, thr):
    src = edge_index[0].astype(jnp.int32)
    dst = edge_index[1].astype(jnp.int32)
    needid = jnp.concatenate([
        jnp.zeros((8,), jnp.int32),
        pos_ind.astype(jnp.int32),
        neg_ind.astype(jnp.int32),
        jnp.zeros((NEED - 264,), jnp.int32),
    ])
    eattr2 = edge_attr.reshape(E // 2, 2 * DE)
    zflag = jnp.zeros((FLAGN,), jnp.int32)
    dstids, xsrc, attrc, xneed, ktot = _compact(dst, src, needid, x, eattr2,
                                                zflag)
    wet = W_e.T
    zpad = jnp.zeros((DE, D), jnp.float32)
    wu = jnp.stack([jnp.concatenate([wet, zpad], axis=0),
                    jnp.concatenate([zpad, wet], axis=0)])
    msg = _edge_mlp(ktot, wu, b_e.reshape(1, D), xsrc, attrc)
    aggrneed = _scatter(msg, dstids, ktot, needid)
    out = _readout(xneed, aggrneed, W1.T, b1.reshape(1, D), W2.T,
                   b2.reshape(1, D), thr)
    return out[0, 0]


# speedup vs baseline: 6.3244x; 1.0769x over previous
import jax
import jax.numpy as jnp
from jax import lax
from jax.experimental import pallas as pl
from jax.experimental.pallas import tpu as pltpu
from jax.experimental.pallas import tpu_sc as plsc

N = 10000
E = 320000
D = 128
DE = 64

NC = 2
NS = 16
ET = E // (NC * NS)
NVEC = ET // 16
FLAGN = 10016
NEED = 384
AGGR_ROWS = 10240
DUMP = N
LCAP = 5120
PHALF = LCAP * NS
CAP = 2 * PHALF
TCB = 2048


def _compact_kernel(dst_h, src_h, needid_h, x_h, zflag_h,
                    dstids_h, xsrc_h, e2c_h, xneed_h, ktot_h,
                    flag_v, dst_v, src_v,
                    loc_eidE, loc_eidO, loc_src, loc_dst, loc_e2,
                    needid_v, cntv, xrows_v, cnt_smem, sem):
    c = lax.axis_index("c")
    s = lax.axis_index("s")
    base = (c * NS + s) * ET

    @pl.when(s == 0)
    def _():
        cnt_smem[0] = 0
        cnt_smem[1] = 0
    plsc.subcore_barrier()

    pltpu.sync_copy(zflag_h, flag_v)
    pltpu.sync_copy(needid_h, needid_v)
    pltpu.sync_copy(dst_h.at[pl.ds(pl.multiple_of(base, 16), ET)], dst_v)
    pltpu.sync_copy(src_h.at[pl.ds(pl.multiple_of(base, 16), ET)], src_v)

    one16 = jnp.ones((16,), jnp.int32)
    dump16 = jnp.full((16,), DUMP, jnp.int32)
    base16 = jnp.full((16,), base, jnp.int32)
    iota16 = lax.iota(jnp.int32, 16)
    lane_even = (iota16 & 1) == 0

    def flag_body(i, _):
        ids = needid_v[pl.ds(i * 16, 16)]
        plsc.store_scatter(flag_v, [ids], one16)
        return 0
    lax.fori_loop(0, NEED // 16, flag_body, 0)

    def scan_body(i, carry):
        wpe, wpo = carry
        d = dst_v[pl.ds(i * 16, 16)]
        f = plsc.load_gather(flag_v, [d])
        m = f > 0
        me = m & lane_even
        mo = m & (~lane_even)
        mie = me.astype(jnp.int32)
        mio = mo.astype(jnp.int32)
        pose = wpe + plsc.cumsum(mie) - 1
        poso = wpo + plsc.cumsum(mio) - 1
        eid = base16 + (i * 16) + iota16
        plsc.store_scatter(loc_eidE, [pose], eid, mask=me)
        plsc.store_scatter(loc_eidO, [poso], eid, mask=mo)
        return wpe + jnp.sum(mie), wpo + jnp.sum(mio)

    kte, kto = lax.fori_loop(0, NVEC, scan_body,
                             (jnp.int32(0), jnp.int32(0)))
    rows_e = ((kte + 127) // 128) * 128
    rows_o = ((kto + 127) // 128) * 128
    offE = plsc.fetch_and_add(cnt_smem.at[0], rows_e, subcore_id=0)
    offO = plsc.fetch_and_add(cnt_smem.at[1], rows_o, subcore_id=0)
    plsc.subcore_barrier()

    @pl.when(s == 0)
    def _():
        tot_e = cnt_smem[0]
        tot_o = cnt_smem[1]
        cntv[...] = jnp.where(iota16 == 0, tot_e,
                              jnp.where(iota16 == 1, tot_o, 0))
        pltpu.sync_copy(cntv, ktot_h.at[pl.ds(pl.multiple_of(c * 16, 16), 16)])

    def emit(loc_eid, k, nrows, goff):
        def pad_eid(j, _):
            pos = k + j * 16 + iota16
            plsc.store_scatter(loc_eid, [pos], base16, mask=pos < nrows)
            return 0
        lax.fori_loop(0, 8, pad_eid, 0)

        def gpass(j, _):
            ev = loc_eid[pl.ds(j * 16, 16)]
            lv = ev - base16
            loc_src[pl.ds(j * 16, 16)] = plsc.load_gather(src_v, [lv])
            loc_dst[pl.ds(j * 16, 16)] = plsc.load_gather(dst_v, [lv])
            loc_e2[pl.ds(j * 16, 16)] = ev >> 1
            return 0
        lax.fori_loop(0, nrows // 16, gpass, 0)

        def pad_dst(j, _):
            pos = k + j * 16 + iota16
            plsc.store_scatter(loc_dst, [pos], dump16, mask=pos < nrows)
            return 0
        lax.fori_loop(0, 8, pad_dst, 0)

        def chunk_body(j, _):
            r = pl.multiple_of(goff + j * 128, 128)
            gr = pl.multiple_of(c * CAP + r, 128)
            pltpu.sync_copy(loc_dst.at[pl.ds(j * 128, 128)],
                            dstids_h.at[pl.ds(gr, 128)])
            pltpu.sync_copy(loc_e2.at[pl.ds(j * 128, 128)],
                            e2c_h.at[pl.ds(gr, 128)])
            pltpu.async_copy(x_h.at[loc_src.at[pl.ds(j * 128, 128)]],
                             xrows_v, sem).wait()
            pltpu.sync_copy(xrows_v, xsrc_h.at[c, pl.ds(r, 128)])
            return 0
        lax.fori_loop(0, nrows // 128, chunk_body, 0)

    emit(loc_eidE, kte, rows_e, offE)
    emit(loc_eidO, kto, rows_o, offO + PHALF)

    @pl.when((c == 0) & (s == 0))
    def _():
        def need_body(j, _):
            pltpu.async_copy(x_h.at[needid_v.at[pl.ds(j * 128, 128)]],
                             xrows_v, sem).wait()
            pltpu.sync_copy(xrows_v, xneed_h.at[pl.ds(j * 128, 128)])
            return 0
        lax.fori_loop(0, NEED // 128, need_body, 0)


def _gattr_kernel(eattr2_h, e2c_h, ktot_h, attrc_h,
                  idxb, arows_v, kv, sem):
    c = lax.axis_index("c")
    s = lax.axis_index("s")
    pltpu.sync_copy(ktot_h.at[pl.ds(pl.multiple_of(c * 16, 16), 16)], kv)
    kvv = kv[pl.ds(0, 16)]
    kte = kvv[0]
    kto = kvv[1]
    nblk_e = (kte + 127) // 128
    nblk = nblk_e + (kto + 127) // 128
    myblk = (nblk - s + (NS - 1)) // NS

    def gb(t, _):
        b = s + NS * t
        r = pl.multiple_of(
            jnp.where(b < nblk_e, b * 128, PHALF + (b - nblk_e) * 128), 128)
        pltpu.sync_copy(
            e2c_h.at[pl.ds(pl.multiple_of(c * CAP + r, 128), 128)], idxb)
        pltpu.async_copy(eattr2_h.at[idxb], arows_v, sem).wait()
        pltpu.sync_copy(arows_v, attrc_h.at[c, pl.ds(r, 128)])
        return 0
    lax.fori_loop(0, myblk, gb, 0)


def _edge_mlp_kernel(ktot_s, wu_v, be_v, xsrc_h, attrc_h, msg_h,
                     xbuf, abuf, mbuf, sem1, sem2, sem3):
    for c in range(NC):
        for par in range(2):
            nblk = (ktot_s[c * 16 + par] + (TCB - 1)) // TCB
            goff = par * PHALF

            def blk(j, _):
                r = goff + j * TCB
                cp1 = pltpu.make_async_copy(xsrc_h.at[c, pl.ds(r, TCB)],
                                            xbuf, sem1)
                cp2 = pltpu.make_async_copy(attrc_h.at[c, pl.ds(r, TCB)],
                                            abuf, sem2)
                cp1.start()
                cp2.start()
                cp1.wait()
                cp2.wait()
                ea = jnp.dot(abuf[...], wu_v[par],
                             preferred_element_type=jnp.float32) + be_v[...]
                mbuf[...] = jnp.maximum(xbuf[...] + ea, 0.0)
                cp3 = pltpu.make_async_copy(mbuf, msg_h.at[c, pl.ds(r, TCB)],
                                            sem3)
                cp3.start()
                cp3.wait()
                return 0
            lax.fori_loop(0, nblk, blk, 0)


def _scatter_kernel(msg_h, dstids_h, ktot_h, needid_h, aggrneed_h,
                    aggr_sh, idxb, mbuf, kv, needid_v, sem):
    c = lax.axis_index("c")
    s = lax.axis_index("s")

    zrow = jnp.zeros((16,), jnp.float32)

    def zb(i, _):
        mbuf[i // 8, pl.ds((i % 8) * 16, 16)] = zrow
        return 0
    lax.fori_loop(0, 128 * 8, zb, 0)

    rows_per_tile = AGGR_ROWS // NS

    def zs(j, _):
        pltpu.sync_copy(mbuf,
                        aggr_sh.at[pl.ds(s * rows_per_tile + j * 128, 128)])
        return 0
    lax.fori_loop(0, rows_per_tile // 128, zs, 0)
    plsc.subcore_barrier()

    pltpu.sync_copy(ktot_h.at[pl.ds(pl.multiple_of(c * 16, 16), 16)], kv)
    kvv = kv[pl.ds(0, 16)]
    kte = kvv[0]
    kto = kvv[1]
    nblk_e = (kte + 127) // 128
    nblk = nblk_e + (kto + 127) // 128
    myblk = (nblk - s + (NS - 1)) // NS

    def sb(t, _):
        b = s + NS * t
        r = pl.multiple_of(
            jnp.where(b < nblk_e, b * 128, PHALF + (b - nblk_e) * 128), 128)
        pltpu.sync_copy(
            dstids_h.at[pl.ds(pl.multiple_of(c * CAP + r, 128), 128)], idxb)
        pltpu.sync_copy(msg_h.at[c, pl.ds(r, 128)], mbuf)
        pltpu.sync_copy(mbuf, aggr_sh.at[idxb], add=True)
        return 0
    lax.fori_loop(0, myblk, sb, 0)
    plsc.subcore_barrier()

    @pl.when(s < NEED // 128)
    def _():
        pltpu.sync_copy(needid_h, needid_v)
        pltpu.async_copy(aggr_sh.at[needid_v.at[pl.ds(s * 128, 128)]],
                         mbuf, sem).wait()
        pltpu.sync_copy(mbuf, aggrneed_h.at[c, pl.ds(s * 128, 128)])


def _readout_kernel(xneed_ref, aggr_ref, w1t_ref, b1_ref, w2t_ref, b2_ref,
                    thr_ref, out_ref):
    h = xneed_ref[...] + aggr_ref[0] + aggr_ref[1]
    pre = jnp.maximum(jnp.dot(h, w1t_ref[...],
                              preferred_element_type=jnp.float32)
                      + b1_ref[...], 0.0)
    emb = jnp.dot(pre, w2t_ref[...],
                  preferred_element_type=jnp.float32) + b2_ref[...]
    nrm = jnp.sqrt(jnp.sum(emb * emb, axis=1, keepdims=True))
    embn = emb / jnp.maximum(nrm, 1e-12)
    center = embn[0:1, :]
    sc = jnp.sum(embn * center, axis=1, keepdims=True)
    r = lax.broadcasted_iota(jnp.int32, (NEED, 1), 0)
    posm = (r >= 8) & (r < 136)
    negm = (r >= 136) & (r < 264)
    pos_mean = jnp.sum(jnp.where(posm, sc, 0.0)) / 128.0
    neg_mean = jnp.sum(jnp.where(negm, sc, 0.0)) / 128.0
    thr = thr_ref[...]
    tn = thr / jnp.maximum(jnp.sqrt(jnp.sum(thr * thr)), 1e-12)
    thr_score = jnp.sum(center * tn)
    pos_loss = jnp.log(jnp.maximum(jax.nn.sigmoid(pos_mean - thr_score),
                                   1e-12))
    neg_loss = jnp.log(jnp.maximum(jax.nn.sigmoid(thr_score - neg_mean),
                                   1e-12))
    out_ref[...] = jnp.full((8, 128), -(pos_loss + neg_loss), jnp.float32)


_sc_mesh = plsc.VectorSubcoreMesh(core_axis_name="c", subcore_axis_name="s",
                                  num_cores=NC, num_subcores=NS)

_compact = pl.kernel(
    _compact_kernel,
    out_type=(
        jax.ShapeDtypeStruct((NC * CAP,), jnp.int32),
        jax.ShapeDtypeStruct((NC, CAP, D), jnp.float32),
        jax.ShapeDtypeStruct((NC * CAP,), jnp.int32),
        jax.ShapeDtypeStruct((NEED, D), jnp.float32),
        jax.ShapeDtypeStruct((NC * 16,), jnp.int32),
    ),
    mesh=_sc_mesh,
    compiler_params=pltpu.CompilerParams(needs_layout_passes=False),
    scratch_types=(
        pltpu.VMEM((FLAGN,), jnp.int32),
        pltpu.VMEM((ET,), jnp.int32),
        pltpu.VMEM((ET,), jnp.int32),
        pltpu.VMEM((LCAP,), jnp.int32),
        pltpu.VMEM((LCAP,), jnp.int32),
        pltpu.VMEM((LCAP,), jnp.int32),
        pltpu.VMEM((LCAP,), jnp.int32),
        pltpu.VMEM((LCAP,), jnp.int32),
        pltpu.VMEM((NEED,), jnp.int32),
        pltpu.VMEM((16,), jnp.int32),
        pltpu.VMEM((128, D), jnp.float32),
        pltpu.SMEM((2,), jnp.int32),
        pltpu.SemaphoreType.DMA,
    ),
)

_gattr = pl.kernel(
    _gattr_kernel,
    out_type=jax.ShapeDtypeStruct((NC, CAP, D), jnp.float32),
    mesh=_sc_mesh,
    compiler_params=pltpu.CompilerParams(needs_layout_passes=False),
    scratch_types=(
        pltpu.VMEM((128,), jnp.int32),
        pltpu.VMEM((128, D), jnp.float32),
        pltpu.VMEM((16,), jnp.int32),
        pltpu.SemaphoreType.DMA,
    ),
)

_scatter = pl.kernel(
    _scatter_kernel,
    out_type=jax.ShapeDtypeStruct((NC, NEED, D), jnp.float32),
    mesh=_sc_mesh,
    compiler_params=pltpu.CompilerParams(needs_layout_passes=False),
    scratch_types=(
        pltpu.VMEM_SHARED((AGGR_ROWS, D), jnp.float32),
        pltpu.VMEM((128,), jnp.int32),
        pltpu.VMEM((128, D), jnp.float32),
        pltpu.VMEM((16,), jnp.int32),
        pltpu.VMEM((NEED,), jnp.int32),
        pltpu.SemaphoreType.DMA,
    ),
)

_edge_mlp = pl.pallas_call(
    _edge_mlp_kernel,
    out_shape=jax.ShapeDtypeStruct((NC, CAP, D), jnp.float32),
    in_specs=[
        pl.BlockSpec(memory_space=pltpu.SMEM),
        pl.BlockSpec(memory_space=pltpu.VMEM),
        pl.BlockSpec(memory_space=pltpu.VMEM),
        pl.BlockSpec(memory_space=pltpu.HBM),
        pl.BlockSpec(memory_space=pltpu.HBM),
    ],
    out_specs=pl.BlockSpec(memory_space=pltpu.HBM),
    scratch_shapes=[
        pltpu.VMEM((TCB, D), jnp.float32),
        pltpu.VMEM((TCB, D), jnp.float32),
        pltpu.VMEM((TCB, D), jnp.float32),
        pltpu.SemaphoreType.DMA,
        pltpu.SemaphoreType.DMA,
        pltpu.SemaphoreType.DMA,
    ],
)

_readout = pl.pallas_call(
    _readout_kernel,
    out_shape=jax.ShapeDtypeStruct((8, 128), jnp.float32),
)


def kernel(x, edge_index, edge_attr, pos_ind, neg_ind, W_e, b_e, W1, b1, W2,
           b2, thr):
    src = edge_index[0].astype(jnp.int32)
    dst = edge_index[1].astype(jnp.int32)
    needid = jnp.concatenate([
        jnp.zeros((8,), jnp.int32),
        pos_ind.astype(jnp.int32),
        neg_ind.astype(jnp.int32),
        jnp.zeros((NEED - 264,), jnp.int32),
    ])
    eattr2 = edge_attr.reshape(E // 2, 2 * DE)
    zflag = jnp.zeros((FLAGN,), jnp.int32)
    dstids, xsrc, e2c, xneed, ktot = _compact(dst, src, needid, x, zflag)
    attrc = _gattr(eattr2, e2c, ktot)
    wet = W_e.T
    zpad = jnp.zeros((DE, D), jnp.float32)
    wu = jnp.stack([jnp.concatenate([wet, zpad], axis=0),
                    jnp.concatenate([zpad, wet], axis=0)])
    msg = _edge_mlp(ktot, wu, b_e.reshape(1, D), xsrc, attrc)
    aggrneed = _scatter(msg, dstids, ktot, needid)
    out = _readout(xneed, aggrneed, W1.T, b1.reshape(1, D), W2.T,
                   b2.reshape(1, D), thr)
    return out[0, 0]
